# Initial kernel scaffold; baseline (speedup 1.0000x reference)
import jax
import jax.numpy as jnp
from jax.experimental import pallas as pl


def kernel(x, triangles, W_node, W_tri):
    raise NotImplementedError("write your pallas kernel here")


# speedup vs baseline: 3.5946x; 3.5946x over previous
import functools

import jax
import jax.numpy as jnp
from jax import lax
from jax.experimental import pallas as pl
from jax.experimental.pallas import tpu as pltpu
from jax.experimental.pallas import tpu_sc as plsc

N_NODES = 50000
N_TRI = 200000
DIM = 128

NC = 2
NS = 16
EW = 16
NE = DIM // EW
PASSES = NE // NC

RPAD = 50048
RT = RPAD // NS
ZR = RT // 8

CHUNK = 128
TPAD = 200704
TPT = TPAD // NS
F_CHUNKS = TPT // CHUNK
CPT = TPAD // (NC * NS)
C_CHUNKS = CPT // CHUNK


def _xp_kernel(x_ref, wn_ref, wt_ref, o_ref):
    wc = jnp.dot(wn_ref[...], wt_ref[...], preferred_element_type=jnp.float32)
    o_ref[...] = jnp.dot(x_ref[...], wc * (1.0 / 3.0),
                         preferred_element_type=jnp.float32)


def _final_kernel(pre_ref, cnt_ref, o_ref):
    cnt = cnt_ref[0, :, 0:1] + cnt_ref[1, :, 0:1]
    v = pre_ref[...] / jnp.maximum(cnt, 1.0)
    o_ref[...] = jnp.where(v > 0.0, v, jnp.exp(v) - 1.0)


def _sc_body(xp, t0, t1, t2, outp, cnt16, slot0, slot1, ibuf, gbuf, sbuf,
             onesb, zbuf):
    c = lax.axis_index("c")
    s = lax.axis_index("s")
    rows0 = s * RT
    tri = (t0, t1, t2)

    def _fill(i, _):
        zbuf[i, :] = jnp.zeros((EW,), jnp.float32)
        return 0

    lax.fori_loop(0, ZR, _fill, 0)

    def _zero_rows(dst):
        for z in range(8):
            pltpu.sync_copy(zbuf, dst.at[pl.ds(rows0 + z * ZR, ZR)])

    def _fill1(i, _):
        onesb[i, :] = jnp.ones((EW,), jnp.float32)
        return 0

    lax.fori_loop(0, CHUNK, _fill1, 0)

    _zero_rows(slot0)
    plsc.subcore_barrier()
    cbase = (c * NS + s) * CPT

    def _cchunk(j, _):
        off = cbase + j * CHUNK
        for k in range(3):
            pltpu.sync_copy(tri[k].at[pl.ds(off, CHUNK)], ibuf.at[k])
        for k in range(3):
            pltpu.sync_copy(onesb, slot0.at[ibuf.at[k]], add=True)
        return 0

    lax.fori_loop(0, C_CHUNKS, _cchunk, 0)
    plsc.subcore_barrier()
    pltpu.sync_copy(slot0.at[pl.ds(rows0, RT)], cnt16.at[c, pl.ds(rows0, RT)])

    for p in range(PASSES):
        e16 = (c * PASSES + p) * EW
        pltpu.sync_copy(xp.at[pl.ds(rows0, RT), pl.ds(e16, EW)],
                        slot0.at[pl.ds(rows0, RT)])
        _zero_rows(slot1)
        plsc.subcore_barrier()
        fbase = s * TPT

        def _fchunk(j, _):
            off = fbase + j * CHUNK
            for k in range(3):
                pltpu.sync_copy(tri[k].at[pl.ds(off, CHUNK)], ibuf.at[k])
            for k in range(3):
                pltpu.sync_copy(slot0.at[ibuf.at[k]], gbuf.at[k])

            def _srow(i, _):
                sbuf[i, :] = gbuf[0, i, :] + gbuf[1, i, :] + gbuf[2, i, :]
                return 0

            lax.fori_loop(0, CHUNK, _srow, 0, unroll=4)
            for k in range(3):
                pltpu.sync_copy(sbuf, slot1.at[ibuf.at[k]], add=True)
            return 0

        lax.fori_loop(0, F_CHUNKS, _fchunk, 0)
        plsc.subcore_barrier()
        pltpu.sync_copy(slot1.at[pl.ds(rows0, RT)],
                        outp.at[pl.ds(rows0, RT), pl.ds(e16, EW)])
        plsc.subcore_barrier()


def kernel(x, triangles, W_node, W_tri):
    nb = N_NODES // 400
    xp = pl.pallas_call(
        _xp_kernel,
        grid=(nb,),
        in_specs=[
            pl.BlockSpec((400, DIM), lambda i: (i, 0)),
            pl.BlockSpec((DIM, DIM), lambda i: (0, 0)),
            pl.BlockSpec((DIM, DIM), lambda i: (0, 0)),
        ],
        out_specs=pl.BlockSpec((400, DIM), lambda i: (i, 0)),
        out_shape=jax.ShapeDtypeStruct((RPAD, DIM), jnp.float32),
    )(x, W_node, W_tri)

    npad = TPAD - N_TRI
    dummy = (N_NODES + (jnp.arange(npad, dtype=jnp.int32) % 16))
    tri_pad = jnp.concatenate(
        [triangles, jnp.broadcast_to(dummy, (3, npad))], axis=1)
    t0, t1, t2 = tri_pad[0], tri_pad[1], tri_pad[2]

    mesh = plsc.VectorSubcoreMesh(core_axis_name="c", subcore_axis_name="s")
    sc_fn = pl.kernel(
        _sc_body,
        out_type=(
            jax.ShapeDtypeStruct((RPAD, DIM), jnp.float32),
            jax.ShapeDtypeStruct((NC, RPAD, EW), jnp.float32),
        ),
        mesh=mesh,
        compiler_params=pltpu.CompilerParams(use_tc_tiling_on_sc=False),
        scratch_types=[
            pltpu.VMEM_SHARED((RPAD, EW), jnp.float32),
            pltpu.VMEM_SHARED((RPAD, EW), jnp.float32),
            pltpu.VMEM((3, CHUNK), jnp.int32),
            pltpu.VMEM((3, CHUNK, EW), jnp.float32),
            pltpu.VMEM((CHUNK, EW), jnp.float32),
            pltpu.VMEM((CHUNK, EW), jnp.float32),
            pltpu.VMEM((ZR, EW), jnp.float32),
        ],
    )
    pre, cnt16 = sc_fn(xp, t0, t1, t2)

    out = pl.pallas_call(
        _final_kernel,
        grid=(nb,),
        in_specs=[
            pl.BlockSpec((400, DIM), lambda i: (i, 0)),
            pl.BlockSpec((NC, 400, EW), lambda i: (0, i, 0)),
        ],
        out_specs=pl.BlockSpec((400, DIM), lambda i: (i, 0)),
        out_shape=jax.ShapeDtypeStruct((N_NODES, DIM), jnp.float32),
    )(pre, cnt16)
    return out


# speedup vs baseline: 8.5594x; 2.3812x over previous
import functools

import jax
import jax.numpy as jnp
from jax import lax
from jax.experimental import pallas as pl
from jax.experimental.pallas import tpu as pltpu
from jax.experimental.pallas import tpu_sc as plsc

N_NODES = 50000
N_TRI = 200000
DIM = 128

NC = 2
NS = 16
EW = 16
NE = DIM // EW
PASSES = NE // NC

RPAD = 50048
RT = RPAD // NS
ZR = RT // 8

CHUNK = 128
TPAD = 204800
TPT = TPAD // NS
F_CHUNKS = TPT // CHUNK
CPT = TPAD // (NC * NS)
C_CHUNKS = CPT // CHUNK


def _xp_kernel(x_ref, wn_ref, wt_ref, o_ref):
    wc = jnp.dot(wn_ref[...], wt_ref[...], preferred_element_type=jnp.float32)
    o_ref[...] = jnp.dot(x_ref[...], wc * (1.0 / 3.0),
                         preferred_element_type=jnp.float32)


def _final_kernel(pre_ref, cnt_ref, o_ref):
    cnt = cnt_ref[0, :, 0:1] + cnt_ref[1, :, 0:1]
    v = pre_ref[...] / jnp.maximum(cnt, 1.0)
    o_ref[...] = jnp.where(v > 0.0, v, jnp.exp(v) - 1.0)


def _sc_body(xp, t0, t1, t2, outp, cnt16, slot0, slot1, ibuf, gbuf, sbuf,
             onesb, zbuf, si0, si1, si2, si3, sg0, sg1, ss0, ss1, st, so):
    si = (si0, si1, si2, si3)
    sg = (sg0, sg1)
    ss = (ss0, ss1)
    tri = (t0, t1, t2)
    c = lax.axis_index("c")
    s = lax.axis_index("s")
    rows0 = s * RT

    def _fill(i, _):
        zbuf[i, :] = jnp.zeros((EW,), jnp.float32)
        return 0

    lax.fori_loop(0, ZR, _fill, 0)

    def _fill1(i, _):
        onesb[i, :] = jnp.ones((EW,), jnp.float32)
        return 0

    lax.fori_loop(0, CHUNK, _fill1, 0)

    def _zero_rows(dst):
        for z in range(8):
            pltpu.sync_copy(zbuf, dst.at[pl.ds(rows0 + z * ZR, ZR)])

    def issue_idx(j, m, base):
        off = base + j * CHUNK
        for k in range(3):
            pltpu.async_copy(tri[k].at[pl.ds(off, CHUNK)], ibuf.at[m, k],
                             si[m])

    def drain_idx(j, m, base):
        off = base + j * CHUNK
        for k in range(3):
            pltpu.make_async_copy(tri[k].at[pl.ds(off, CHUNK)], ibuf.at[m, k],
                                  si[m]).wait()

    def issue_gather(m, g):
        for k in range(3):
            pltpu.async_copy(slot0.at[ibuf.at[m, k]], gbuf.at[g, k], sg[g])

    def drain_gather(m, g):
        for k in range(3):
            pltpu.make_async_copy(slot0.at[ibuf.at[m, k]], gbuf.at[g, k],
                                  sg[g]).wait()

    def issue_scatter(g, m):
        for k in range(3):
            pltpu.async_copy(sbuf.at[g], slot1.at[ibuf.at[m, k]], ss[g],
                             add=True)

    def drain_scatter(g, m):
        for k in range(3):
            pltpu.make_async_copy(sbuf.at[g], slot1.at[ibuf.at[m, k]],
                                  ss[g]).wait()

    def c_issue_scatter(u):
        for k in range(3):
            pltpu.async_copy(onesb, slot0.at[ibuf.at[u, k]], ss[u], add=True)

    def c_drain_scatter(u):
        for k in range(3):
            pltpu.make_async_copy(onesb, slot0.at[ibuf.at[u, k]],
                                  ss[u]).wait()

    _zero_rows(slot0)
    plsc.subcore_barrier()
    cbase = (c * NS + s) * CPT
    issue_idx(0, 0, cbase)

    def _cgrp(i, _):
        for u in range(2):
            j = 2 * i + u
            drain_idx(j, u, cbase)

            @pl.when(j >= 1)
            def _():
                c_drain_scatter(1 - u)

            c_issue_scatter(u)

            @pl.when(j + 1 < C_CHUNKS)
            def _():
                issue_idx(j + 1, 1 - u, cbase)

        return 0

    lax.fori_loop(0, C_CHUNKS // 2, _cgrp, 0)
    c_drain_scatter(1)
    plsc.subcore_barrier()
    pltpu.async_copy(slot0.at[pl.ds(rows0, RT)], cnt16.at[c, pl.ds(rows0, RT)],
                     so)

    fbase = s * TPT
    for p in range(PASSES):
        e16 = (c * PASSES + p) * EW
        if p == 0:
            pltpu.make_async_copy(slot0.at[pl.ds(rows0, RT)],
                                  cnt16.at[c, pl.ds(rows0, RT)], so).wait()
        pltpu.async_copy(xp.at[pl.ds(rows0, RT), pl.ds(e16, EW)],
                         slot0.at[pl.ds(rows0, RT)], st)
        if p > 0:
            pe16 = (c * PASSES + (p - 1)) * EW
            pltpu.make_async_copy(slot1.at[pl.ds(rows0, RT)],
                                  outp.at[pl.ds(rows0, RT), pl.ds(pe16, EW)],
                                  so).wait()
        _zero_rows(slot1)
        pltpu.make_async_copy(xp.at[pl.ds(rows0, RT), pl.ds(e16, EW)],
                              slot0.at[pl.ds(rows0, RT)], st).wait()
        plsc.subcore_barrier()

        issue_idx(0, 0, fbase)
        issue_idx(1, 1, fbase)
        drain_idx(0, 0, fbase)
        issue_gather(0, 0)

        def _fgrp(i, _):
            for u in range(4):
                j = 4 * i + u
                g = u % 2
                mp1 = (u + 1) % 4
                mp2 = (u + 2) % 4
                gp1 = (u + 1) % 2

                @pl.when(j >= 2)
                def _():
                    drain_scatter(g, mp2)

                drain_gather(u, g)

                @pl.when(j + 2 < F_CHUNKS)
                def _():
                    issue_idx(j + 2, mp2, fbase)

                @pl.when(j + 1 < F_CHUNKS)
                def _():
                    drain_idx(j + 1, mp1, fbase)
                    issue_gather(mp1, gp1)

                def _srow(i2, _2):
                    sbuf[g, i2, :] = (gbuf[g, 0, i2, :] + gbuf[g, 1, i2, :] +
                                      gbuf[g, 2, i2, :])
                    return 0

                lax.fori_loop(0, CHUNK, _srow, 0, unroll=4)
                issue_scatter(g, u)
            return 0

        lax.fori_loop(0, F_CHUNKS // 4, _fgrp, 0)
        drain_scatter(0, 2)
        drain_scatter(1, 3)
        plsc.subcore_barrier()
        pltpu.async_copy(slot1.at[pl.ds(rows0, RT)],
                         outp.at[pl.ds(rows0, RT), pl.ds(e16, EW)], so)

    le16 = (c * PASSES + (PASSES - 1)) * EW
    pltpu.make_async_copy(slot1.at[pl.ds(rows0, RT)],
                          outp.at[pl.ds(rows0, RT), pl.ds(le16, EW)],
                          so).wait()


def kernel(x, triangles, W_node, W_tri):
    nb = N_NODES // 400
    xp = pl.pallas_call(
        _xp_kernel,
        grid=(nb,),
        in_specs=[
            pl.BlockSpec((400, DIM), lambda i: (i, 0)),
            pl.BlockSpec((DIM, DIM), lambda i: (0, 0)),
            pl.BlockSpec((DIM, DIM), lambda i: (0, 0)),
        ],
        out_specs=pl.BlockSpec((400, DIM), lambda i: (i, 0)),
        out_shape=jax.ShapeDtypeStruct((RPAD, DIM), jnp.float32),
    )(x, W_node, W_tri)

    npad = TPAD - N_TRI
    dummy = (N_NODES + (jnp.arange(npad, dtype=jnp.int32) % 16))
    tri_pad = jnp.concatenate(
        [triangles, jnp.broadcast_to(dummy, (3, npad))], axis=1)
    t0, t1, t2 = tri_pad[0], tri_pad[1], tri_pad[2]

    mesh = plsc.VectorSubcoreMesh(core_axis_name="c", subcore_axis_name="s")
    sc_fn = pl.kernel(
        _sc_body,
        out_type=(
            jax.ShapeDtypeStruct((RPAD, DIM), jnp.float32),
            jax.ShapeDtypeStruct((NC, RPAD, EW), jnp.float32),
        ),
        mesh=mesh,
        compiler_params=pltpu.CompilerParams(use_tc_tiling_on_sc=False),
        scratch_types=[
            pltpu.VMEM_SHARED((RPAD, EW), jnp.float32),
            pltpu.VMEM_SHARED((RPAD, EW), jnp.float32),
            pltpu.VMEM((4, 3, CHUNK), jnp.int32),
            pltpu.VMEM((2, 3, CHUNK, EW), jnp.float32),
            pltpu.VMEM((2, CHUNK, EW), jnp.float32),
            pltpu.VMEM((CHUNK, EW), jnp.float32),
            pltpu.VMEM((ZR, EW), jnp.float32),
            pltpu.SemaphoreType.DMA,
            pltpu.SemaphoreType.DMA,
            pltpu.SemaphoreType.DMA,
            pltpu.SemaphoreType.DMA,
            pltpu.SemaphoreType.DMA,
            pltpu.SemaphoreType.DMA,
            pltpu.SemaphoreType.DMA,
            pltpu.SemaphoreType.DMA,
            pltpu.SemaphoreType.DMA,
            pltpu.SemaphoreType.DMA,
        ],
    )
    pre, cnt16 = sc_fn(xp, t0, t1, t2)

    out = pl.pallas_call(
        _final_kernel,
        grid=(nb,),
        in_specs=[
            pl.BlockSpec((400, DIM), lambda i: (i, 0)),
            pl.BlockSpec((NC, 400, EW), lambda i: (0, i, 0)),
        ],
        out_specs=pl.BlockSpec((400, DIM), lambda i: (i, 0)),
        out_shape=jax.ShapeDtypeStruct((N_NODES, DIM), jnp.float32),
    )(pre, cnt16)
    return out


# speedup vs baseline: 11.3038x; 1.3206x over previous
import functools

import jax
import jax.numpy as jnp
from jax import lax
from jax.experimental import pallas as pl
from jax.experimental.pallas import tpu as pltpu
from jax.experimental.pallas import tpu_sc as plsc

N_NODES = 50000
N_TRI = 200000
DIM = 128

NC = 2
NS = 16
QW = 32
NQ = DIM // QW
PASSES = NQ // NC
CW = 16

RPAD = 50016
RT = RPAD // NS
ZR = RT // 6

CHUNK = 128
TPAD = 204800
TROWS = TPAD // CHUNK
TPT = TPAD // NS
F_CHUNKS = TPT // CHUNK
F_BATCH = 2
CPT = TPAD // (NC * NS)
C_CHUNKS = CPT // CHUNK
C_BATCH = 5


def _xp_kernel(x_ref, wn_ref, wt_ref, o_ref):
    wc = jnp.dot(wn_ref[...], wt_ref[...], preferred_element_type=jnp.float32)
    o_ref[...] = jnp.dot(x_ref[...], wc * (1.0 / 3.0),
                         preferred_element_type=jnp.float32)


def _final_kernel(pre_ref, cnt_ref, o_ref):
    cnt = cnt_ref[0, :, 0:1] + cnt_ref[1, :, 0:1]
    v = pre_ref[...] / jnp.maximum(cnt, 1.0)
    o_ref[...] = jnp.where(v > 0.0, v, jnp.exp(v) - 1.0)


def _counts_body(tri, cnt16, cacc, ibuf, onesb, zbuf, si, ss0, ss1):
    ss = (ss0, ss1)
    c = lax.axis_index("c")
    s = lax.axis_index("s")
    rows0 = s * RT

    def _fill1(i, _):
        onesb[i, :] = jnp.ones((CW,), jnp.float32)
        return 0

    lax.fori_loop(0, CHUNK, _fill1, 0)

    def _fillz(i, _):
        zbuf[i, :] = jnp.zeros((CW,), jnp.float32)
        return 0

    lax.fori_loop(0, ZR, _fillz, 0)
    for z in range(6):
        pltpu.sync_copy(zbuf, cacc.at[pl.ds(rows0 + z * ZR, ZR)])
    plsc.subcore_barrier()

    crow0 = (c * NS + s) * (CPT // CHUNK)

    def issue_idx(bi, sl):
        pltpu.async_copy(tri.at[pl.ds(crow0 + bi * C_BATCH, C_BATCH)],
                         ibuf.at[sl], si)

    def drain_idx(bi, sl):
        pltpu.make_async_copy(tri.at[pl.ds(crow0 + bi * C_BATCH, C_BATCH)],
                              ibuf.at[sl], si).wait()

    def issue_sc(sl, m, g):
        for k in range(3):
            pltpu.async_copy(onesb, cacc.at[ibuf.at[sl, m, k]], ss[g],
                             add=True)

    def drain_sc(sl, m, g):
        for k in range(3):
            pltpu.make_async_copy(onesb, cacc.at[ibuf.at[sl, m, k]],
                                  ss[g]).wait()

    issue_idx(0, 0)
    drain_idx(0, 0)

    def _cgrp(i, _):
        for u2 in range(2):
            bi = 2 * i + u2
            for m in range(C_BATCH):
                j = bi * C_BATCH + m
                g = (u2 + m) % 2

                @pl.when(j >= 2)
                def _():
                    if m >= 2:
                        drain_sc(u2, m - 2, g)
                    else:
                        drain_sc(1 - u2, m + 3, g)

                if m == 2:
                    @pl.when(bi + 1 < 2 * (C_CHUNKS // (2 * C_BATCH)))
                    def _():
                        issue_idx(bi + 1, 1 - u2)

                if m == 4:
                    @pl.when(j + 1 < C_CHUNKS)
                    def _():
                        drain_idx(bi + 1, 1 - u2)

                issue_sc(u2, m, g)
        return 0

    lax.fori_loop(0, C_CHUNKS // (2 * C_BATCH), _cgrp, 0)
    drain_sc(1, 3, 0)
    drain_sc(1, 4, 1)
    plsc.subcore_barrier()
    pltpu.sync_copy(cacc.at[pl.ds(rows0, RT)], cnt16.at[c, pl.ds(rows0, RT)])


def _main_body(xp4, tri, zeros, outp, acc, ibuf, gidx, gbuf, si, sg0, sg1,
               ss0, ss1, st, so):
    sg = (sg0, sg1)
    ss = (ss0, ss1)
    c = lax.axis_index("c")
    s = lax.axis_index("s")
    rows0 = s * RT
    frow0 = s * (TPT // CHUNK)

    def issue_idx(bi, sl):
        pltpu.async_copy(tri.at[pl.ds(frow0 + bi * F_BATCH, F_BATCH)],
                         ibuf.at[sl], si)

    def drain_idx(bi, sl):
        pltpu.make_async_copy(tri.at[pl.ds(frow0 + bi * F_BATCH, F_BATCH)],
                              ibuf.at[sl], si).wait()

    def compute_gidx(sl, q):
        base = q
        def _gi(t, _):
            for m2 in range(F_BATCH):
                for k in range(3):
                    v = ibuf[sl, m2, k, pl.ds(t * 16, 16)]
                    gidx[m2, k, pl.ds(t * 16, 16)] = (v << 2) + base
            return 0
        lax.fori_loop(0, CHUNK // 16, _gi, 0, unroll=2)

    def issue_gather(m, g):
        for k in range(3):
            pltpu.async_copy(xp4.at[gidx.at[m, k]], gbuf.at[g, k], sg[g])

    def drain_gather(m, g):
        for k in range(3):
            pltpu.make_async_copy(xp4.at[gidx.at[m, k]], gbuf.at[g, k],
                                  sg[g]).wait()

    def issue_scatter(sl, m, g):
        for k in range(3):
            pltpu.async_copy(gbuf.at[g, 0], acc.at[ibuf.at[sl, m, k]], ss[g],
                             add=True)

    def drain_scatter(sl, m, g):
        for k in range(3):
            pltpu.make_async_copy(gbuf.at[g, 0], acc.at[ibuf.at[sl, m, k]],
                                  ss[g]).wait()

    for p in range(PASSES):
        q = c * PASSES + p
        if p > 0:
            pq = q - 1
            pltpu.make_async_copy(
                acc.at[pl.ds(rows0, RT)],
                outp.at[pl.ds(rows0, RT), pl.ds(pq * QW, QW)], so).wait()
        pltpu.async_copy(zeros, acc.at[pl.ds(rows0, RT)], st)
        issue_idx(0, 0)
        drain_idx(0, 0)
        compute_gidx(0, q)
        pltpu.make_async_copy(zeros, acc.at[pl.ds(rows0, RT)], st).wait()
        plsc.subcore_barrier()
        issue_gather(0, 0)

        def _fgrp(i, _):
            for u2 in range(2):
                bi = 2 * i + u2
                for m in range(F_BATCH):
                    j = bi * F_BATCH + m
                    g = m

                    @pl.when(j >= 1)
                    def _():
                        if m == 1:
                            drain_scatter(u2, 0, 1 - g)
                        else:
                            drain_scatter(1 - u2, 1, 1 - g)

                    drain_gather(m, g)

                    if m == 0:
                        @pl.when(bi + 1 < F_CHUNKS // F_BATCH)
                        def _():
                            issue_idx(bi + 1, 1 - u2)

                        issue_gather(1, 1)
                    else:
                        @pl.when(j + 1 < F_CHUNKS)
                        def _():
                            drain_idx(bi + 1, 1 - u2)
                            compute_gidx(1 - u2, q)
                            issue_gather(0, 0)

                    def _srow(i2, _2):
                        gbuf[g, 0, i2, :] = (gbuf[g, 0, i2, :] +
                                             gbuf[g, 1, i2, :] +
                                             gbuf[g, 2, i2, :])
                        return 0

                    lax.fori_loop(0, CHUNK, _srow, 0, unroll=4)
                    issue_scatter(u2, m, g)
            return 0

        lax.fori_loop(0, F_CHUNKS // (2 * F_BATCH), _fgrp, 0)
        drain_scatter(1, 1, 1)
        plsc.subcore_barrier()
        pltpu.async_copy(acc.at[pl.ds(rows0, RT)],
                         outp.at[pl.ds(rows0, RT), pl.ds(q * QW, QW)], so)

    lq = c * PASSES + (PASSES - 1)
    pltpu.make_async_copy(acc.at[pl.ds(rows0, RT)],
                          outp.at[pl.ds(rows0, RT), pl.ds(lq * QW, QW)],
                          so).wait()


def kernel(x, triangles, W_node, W_tri):
    nb = N_NODES // 400
    xp = pl.pallas_call(
        _xp_kernel,
        grid=(nb,),
        in_specs=[
            pl.BlockSpec((400, DIM), lambda i: (i, 0)),
            pl.BlockSpec((DIM, DIM), lambda i: (0, 0)),
            pl.BlockSpec((DIM, DIM), lambda i: (0, 0)),
        ],
        out_specs=pl.BlockSpec((400, DIM), lambda i: (i, 0)),
        out_shape=jax.ShapeDtypeStruct((RPAD, DIM), jnp.float32),
    )(x, W_node, W_tri)

    npad = TPAD - N_TRI
    dummy = (N_NODES + (jnp.arange(npad, dtype=jnp.int32) % 16))
    tri_pad = jnp.concatenate(
        [triangles, jnp.broadcast_to(dummy, (3, npad))], axis=1)
    tri_c = jnp.transpose(tri_pad.reshape(3, TROWS, CHUNK), (1, 0, 2))

    mesh = plsc.VectorSubcoreMesh(core_axis_name="c", subcore_axis_name="s")

    counts_fn = pl.kernel(
        _counts_body,
        out_type=jax.ShapeDtypeStruct((NC, RPAD, CW), jnp.float32),
        mesh=mesh,
        compiler_params=pltpu.CompilerParams(use_tc_tiling_on_sc=False),
        scratch_types=[
            pltpu.VMEM_SHARED((RPAD, CW), jnp.float32),
            pltpu.VMEM((2, C_BATCH, 3, CHUNK), jnp.int32),
            pltpu.VMEM((CHUNK, CW), jnp.float32),
            pltpu.VMEM((ZR, CW), jnp.float32),
            pltpu.SemaphoreType.DMA,
            pltpu.SemaphoreType.DMA,
            pltpu.SemaphoreType.DMA,
        ],
    )
    cnt16 = counts_fn(tri_c)

    zeros = jnp.zeros((RT, QW), jnp.float32)
    main_fn = pl.kernel(
        _main_body,
        out_type=jax.ShapeDtypeStruct((RPAD, DIM), jnp.float32),
        mesh=mesh,
        compiler_params=pltpu.CompilerParams(use_tc_tiling_on_sc=False),
        scratch_types=[
            pltpu.VMEM_SHARED((RPAD, QW), jnp.float32),
            pltpu.VMEM((2, F_BATCH, 3, CHUNK), jnp.int32),
            pltpu.VMEM((F_BATCH, 3, CHUNK), jnp.int32),
            pltpu.VMEM((2, 3, CHUNK, QW), jnp.float32),
            pltpu.SemaphoreType.DMA,
            pltpu.SemaphoreType.DMA,
            pltpu.SemaphoreType.DMA,
            pltpu.SemaphoreType.DMA,
            pltpu.SemaphoreType.DMA,
            pltpu.SemaphoreType.DMA,
            pltpu.SemaphoreType.DMA,
        ],
    )
    pre = main_fn(xp.reshape(NQ * RPAD, QW), tri_c, zeros)

    out = pl.pallas_call(
        _final_kernel,
        grid=(nb,),
        in_specs=[
            pl.BlockSpec((400, DIM), lambda i: (i, 0)),
            pl.BlockSpec((NC, 400, CW), lambda i: (0, i, 0)),
        ],
        out_specs=pl.BlockSpec((400, DIM), lambda i: (i, 0)),
        out_shape=jax.ShapeDtypeStruct((N_NODES, DIM), jnp.float32),
    )(pre, cnt16)
    return out


# speedup vs baseline: 13.2433x; 1.1716x over previous
import functools

import jax
import jax.numpy as jnp
from jax import lax
from jax.experimental import pallas as pl
from jax.experimental.pallas import tpu as pltpu
from jax.experimental.pallas import tpu_sc as plsc

N_NODES = 50000
N_TRI = 200000
DIM = 128

NC = 2
NS = 16
QW = 32
NQ = DIM // QW
PASSES = NQ // NC
CW = 16

RPAD = 50016
RT = RPAD // NS
ZR = RT // 6

CHUNK = 128
TPAD = 204800
TROWS = TPAD // CHUNK
TPT = TPAD // NS
F_CHUNKS = TPT // CHUNK
F_BATCH = 2
CPT = TPAD // (NC * NS)
C_CHUNKS = CPT // CHUNK
C_BATCH = 5


def _xp_kernel(x_ref, wn_ref, wt_ref, o_ref):
    wc = jnp.dot(wn_ref[...], wt_ref[...], preferred_element_type=jnp.float32)
    o_ref[...] = jnp.dot(x_ref[...], wc * (1.0 / 3.0),
                         preferred_element_type=jnp.float32)


def _final_kernel(pre_ref, cnt_ref, o_ref):
    cnt = cnt_ref[0, :, 0:1] + cnt_ref[1, :, 0:1]
    v = pre_ref[...] / jnp.maximum(cnt, 1.0)
    o_ref[...] = jnp.where(v > 0.0, v, jnp.exp(v) - 1.0)


def _counts_body(tri, cnt16, cacc, ibuf, onesb, zbuf, si, ss0, ss1):
    ss = (ss0, ss1)
    c = lax.axis_index("c")
    s = lax.axis_index("s")
    rows0 = s * RT

    def _fill1(i, _):
        onesb[i, :] = jnp.ones((CW,), jnp.float32)
        return 0

    lax.fori_loop(0, CHUNK, _fill1, 0)

    def _fillz(i, _):
        zbuf[i, :] = jnp.zeros((CW,), jnp.float32)
        return 0

    lax.fori_loop(0, ZR, _fillz, 0)
    for z in range(6):
        pltpu.sync_copy(zbuf, cacc.at[pl.ds(rows0 + z * ZR, ZR)])
    plsc.subcore_barrier()

    crow0 = (c * NS + s) * (CPT // CHUNK)

    def issue_idx(bi, sl):
        pltpu.async_copy(tri.at[pl.ds(crow0 + bi * C_BATCH, C_BATCH)],
                         ibuf.at[sl], si)

    def drain_idx(bi, sl):
        pltpu.make_async_copy(tri.at[pl.ds(crow0 + bi * C_BATCH, C_BATCH)],
                              ibuf.at[sl], si).wait()

    def issue_sc(sl, m, g):
        for k in range(3):
            pltpu.async_copy(onesb, cacc.at[ibuf.at[sl, m, k]], ss[g],
                             add=True)

    def drain_sc(sl, m, g):
        for k in range(3):
            pltpu.make_async_copy(onesb, cacc.at[ibuf.at[sl, m, k]],
                                  ss[g]).wait()

    issue_idx(0, 0)
    drain_idx(0, 0)

    def _cgrp(i, _):
        for u2 in range(2):
            bi = 2 * i + u2
            for m in range(C_BATCH):
                j = bi * C_BATCH + m
                g = (u2 + m) % 2

                @pl.when(j >= 2)
                def _():
                    if m >= 2:
                        drain_sc(u2, m - 2, g)
                    else:
                        drain_sc(1 - u2, m + 3, g)

                if m == 2:
                    @pl.when(bi + 1 < 2 * (C_CHUNKS // (2 * C_BATCH)))
                    def _():
                        issue_idx(bi + 1, 1 - u2)

                if m == 4:
                    @pl.when(j + 1 < C_CHUNKS)
                    def _():
                        drain_idx(bi + 1, 1 - u2)

                issue_sc(u2, m, g)
        return 0

    lax.fori_loop(0, C_CHUNKS // (2 * C_BATCH), _cgrp, 0)
    drain_sc(1, 3, 0)
    drain_sc(1, 4, 1)
    plsc.subcore_barrier()
    pltpu.sync_copy(cacc.at[pl.ds(rows0, RT)], cnt16.at[c, pl.ds(rows0, RT)])


def _main_body(xp4, tri, zeros, outp, acc, ibuf, gidx, gbuf, si, sg0, sg1,
               ss0, ss1, st, so):
    sg = (sg0, sg1)
    ss = (ss0, ss1)
    c = lax.axis_index("c")
    s = lax.axis_index("s")
    rows0 = s * RT
    frow0 = s * (TPT // CHUNK)

    def issue_idx(bi, sl):
        pltpu.async_copy(tri.at[pl.ds(frow0 + bi * F_BATCH, F_BATCH)],
                         ibuf.at[sl], si)

    def drain_idx(bi, sl):
        pltpu.make_async_copy(tri.at[pl.ds(frow0 + bi * F_BATCH, F_BATCH)],
                              ibuf.at[sl], si).wait()

    def compute_gidx(sl, q):
        base = q
        def _gi(t, _):
            for m2 in range(F_BATCH):
                for k in range(3):
                    v = ibuf[sl, m2, k, pl.ds(t * 16, 16)]
                    gidx[m2, k, pl.ds(t * 16, 16)] = (v << 2) + base
            return 0
        lax.fori_loop(0, CHUNK // 16, _gi, 0, unroll=2)

    def issue_gather(m, g):
        for k in range(3):
            pltpu.async_copy(xp4.at[gidx.at[m, k]], gbuf.at[g, k], sg[g])

    def drain_gather(m, g):
        for k in range(3):
            pltpu.make_async_copy(xp4.at[gidx.at[m, k]], gbuf.at[g, k],
                                  sg[g]).wait()

    def issue_scatter(sl, m, g):
        for k in range(3):
            pltpu.async_copy(gbuf.at[g, 0], acc.at[ibuf.at[sl, m, k]], ss[g],
                             add=True)

    def drain_scatter(sl, m, g):
        for k in range(3):
            pltpu.make_async_copy(gbuf.at[g, 0], acc.at[ibuf.at[sl, m, k]],
                                  ss[g]).wait()

    for p in range(PASSES):
        q = c * PASSES + p
        if p > 0:
            pq = q - 1
            pltpu.make_async_copy(
                acc.at[pl.ds(rows0, RT)],
                outp.at[pl.ds(rows0, RT), pl.ds(pq * QW, QW)], so).wait()
        pltpu.async_copy(zeros, acc.at[pl.ds(rows0, RT)], st)
        issue_idx(0, 0)
        drain_idx(0, 0)
        compute_gidx(0, q)
        pltpu.make_async_copy(zeros, acc.at[pl.ds(rows0, RT)], st).wait()
        plsc.subcore_barrier()
        issue_gather(0, 0)

        def _fgrp(i, _):
            for u2 in range(2):
                bi = 2 * i + u2
                for m in range(F_BATCH):
                    j = bi * F_BATCH + m
                    g = m

                    @pl.when(j >= 1)
                    def _():
                        if m == 1:
                            drain_scatter(u2, 0, 1 - g)
                        else:
                            drain_scatter(1 - u2, 1, 1 - g)

                    drain_gather(m, g)

                    if m == 0:
                        @pl.when(bi + 1 < F_CHUNKS // F_BATCH)
                        def _():
                            issue_idx(bi + 1, 1 - u2)

                        issue_gather(1, 1)
                    else:
                        @pl.when(j + 1 < F_CHUNKS)
                        def _():
                            drain_idx(bi + 1, 1 - u2)
                            compute_gidx(1 - u2, q)
                            issue_gather(0, 0)

                    def _srow(i2, _2):
                        gbuf[g, 0, i2, :] = (gbuf[g, 0, i2, :] +
                                             gbuf[g, 1, i2, :] +
                                             gbuf[g, 2, i2, :])
                        return 0

                    lax.fori_loop(0, CHUNK, _srow, 0, unroll=4)
                    issue_scatter(u2, m, g)
            return 0

        lax.fori_loop(0, F_CHUNKS // (2 * F_BATCH), _fgrp, 0)
        drain_scatter(1, 1, 1)
        plsc.subcore_barrier()
        pltpu.async_copy(acc.at[pl.ds(rows0, RT)],
                         outp.at[pl.ds(rows0, RT), pl.ds(q * QW, QW)], so)

    lq = c * PASSES + (PASSES - 1)
    pltpu.make_async_copy(acc.at[pl.ds(rows0, RT)],
                          outp.at[pl.ds(rows0, RT), pl.ds(lq * QW, QW)],
                          so).wait()


def kernel(x, triangles, W_node, W_tri):
    nb = N_NODES // 1000
    xp = pl.pallas_call(
        _xp_kernel,
        grid=(nb,),
        in_specs=[
            pl.BlockSpec((1000, DIM), lambda i: (i, 0)),
            pl.BlockSpec((DIM, DIM), lambda i: (0, 0)),
            pl.BlockSpec((DIM, DIM), lambda i: (0, 0)),
        ],
        out_specs=pl.BlockSpec((1000, DIM), lambda i: (i, 0)),
        out_shape=jax.ShapeDtypeStruct((RPAD, DIM), jnp.float32),
    )(x, W_node, W_tri)

    npad = TPAD - N_TRI
    dummy = (N_NODES + (jnp.arange(npad, dtype=jnp.int32) % 16))
    tri_pad = jnp.concatenate(
        [triangles, jnp.broadcast_to(dummy, (3, npad))], axis=1)
    tri_c = jnp.transpose(tri_pad.reshape(3, TROWS, CHUNK), (1, 0, 2))

    mesh = plsc.VectorSubcoreMesh(core_axis_name="c", subcore_axis_name="s")

    counts_fn = pl.kernel(
        _counts_body,
        out_type=jax.ShapeDtypeStruct((NC, RPAD, CW), jnp.float32),
        mesh=mesh,
        compiler_params=pltpu.CompilerParams(use_tc_tiling_on_sc=False),
        scratch_types=[
            pltpu.VMEM_SHARED((RPAD, CW), jnp.float32),
            pltpu.VMEM((2, C_BATCH, 3, CHUNK), jnp.int32),
            pltpu.VMEM((CHUNK, CW), jnp.float32),
            pltpu.VMEM((ZR, CW), jnp.float32),
            pltpu.SemaphoreType.DMA,
            pltpu.SemaphoreType.DMA,
            pltpu.SemaphoreType.DMA,
        ],
    )
    cnt16 = counts_fn(tri_c)

    zeros = jnp.zeros((RT, QW), jnp.float32)
    main_fn = pl.kernel(
        _main_body,
        out_type=jax.ShapeDtypeStruct((RPAD, DIM), jnp.float32),
        mesh=mesh,
        compiler_params=pltpu.CompilerParams(use_tc_tiling_on_sc=False),
        scratch_types=[
            pltpu.VMEM_SHARED((RPAD, QW), jnp.float32),
            pltpu.VMEM((2, F_BATCH, 3, CHUNK), jnp.int32),
            pltpu.VMEM((F_BATCH, 3, CHUNK), jnp.int32),
            pltpu.VMEM((2, 3, CHUNK, QW), jnp.float32),
            pltpu.SemaphoreType.DMA,
            pltpu.SemaphoreType.DMA,
            pltpu.SemaphoreType.DMA,
            pltpu.SemaphoreType.DMA,
            pltpu.SemaphoreType.DMA,
            pltpu.SemaphoreType.DMA,
            pltpu.SemaphoreType.DMA,
        ],
    )
    pre = main_fn(xp.reshape(NQ * RPAD, QW), tri_c, zeros)

    out = pl.pallas_call(
        _final_kernel,
        grid=(nb,),
        in_specs=[
            pl.BlockSpec((1000, DIM), lambda i: (i, 0)),
            pl.BlockSpec((NC, 1000, CW), lambda i: (0, i, 0)),
        ],
        out_specs=pl.BlockSpec((1000, DIM), lambda i: (i, 0)),
        out_shape=jax.ShapeDtypeStruct((N_NODES, DIM), jnp.float32),
    )(pre, cnt16)
    return out
